# baseline (speedup 1.0000x reference)
import functools

import jax
import jax.numpy as jnp
from jax import lax
from jax.experimental import pallas as pl
from jax.experimental.pallas import tpu as pltpu
from jax.experimental.pallas import tpu_sc as plsc

N = 10000
E = 320000
D1 = 128
D2 = 256
G = 64

NC = 2
NS = 16
CHUNK = 80
NPAD = 10240
ROWS_PER_TILE = NPAD // NS
ZROWS = 32

def _vmesh():
    return plsc.VectorSubcoreMesh(core_axis_name="c", subcore_axis_name="s")


def _zero_fill(buf):
    @pl.loop(0, CHUNK)
    def _(i):
        for j in range(8):
            buf[i, pl.ds(j * 16, 16)] = jnp.zeros((16,), jnp.float32)


NSETS = 3


def _scatter_chunks(table, src_flat, dst_flat, base, acc, isrc, idst, rows,
                    sem_i, sem_g, sem_s, nchunks):
    def idx_load(c, j):
        off = base + c * CHUNK
        pltpu.async_copy(src_flat.at[pl.ds(off, CHUNK)], isrc[j], sem_i[j])
        pltpu.async_copy(dst_flat.at[pl.ds(off, CHUNK)], idst[j], sem_i[j])

    def idx_wait(j):
        pltpu.make_async_copy(src_flat.at[pl.ds(0, CHUNK)], isrc[j],
                              sem_i[j]).wait()
        pltpu.make_async_copy(dst_flat.at[pl.ds(0, CHUNK)], idst[j],
                              sem_i[j]).wait()

    def gather_wait(j):
        pltpu.make_async_copy(table.at[isrc[j]], rows[j], sem_g[j]).wait()

    def scatter_start(j):
        pltpu.async_copy(rows[j], acc.at[idst[j]], sem_s[j], add=True)

    def scatter_wait(j):
        pltpu.make_async_copy(rows[j], acc.at[idst[j]], sem_s[j]).wait()

    nrounds = nchunks // NSETS
    tail = nchunks % NSETS

    for j in range(NSETS):
        idx_load(j, j)
    for j in range(NSETS):
        idx_wait(j)
        pltpu.async_copy(table.at[isrc[j]], rows[j], sem_g[j])

    @pl.loop(0, nrounds)
    def _(r):
        c0 = r * NSETS
        for j in range(NSETS):
            gather_wait(j)
            scatter_start(j)
        for j in range(NSETS):
            scatter_wait(j)

            @pl.when(c0 + NSETS + j < nchunks)
            def _():
                idx_load(c0 + NSETS + j, j)
        for j in range(NSETS):

            @pl.when(c0 + NSETS + j < nchunks)
            def _():
                idx_wait(j)
                pltpu.async_copy(table.at[isrc[j]], rows[j], sem_g[j])

    for j in range(tail):
        gather_wait(j)
        scatter_start(j)
    for j in range(tail):
        scatter_wait(j)


def _sc_prologue(acc, rows0, sid):
    _zero_fill(rows0)
    row0 = sid * ROWS_PER_TILE
    for kk in range(ROWS_PER_TILE // CHUNK):
        pltpu.sync_copy(rows0, acc.at[pl.ds(row0 + kk * CHUNK, CHUNK)])


def _sc_epilogue(acc, rows0, out0, out1, cid, sid):
    plsc.subcore_barrier()
    row0 = sid * ROWS_PER_TILE
    for k in range(ROWS_PER_TILE // CHUNK):
        sl = pl.ds(row0 + k * CHUNK, CHUNK)
        pltpu.sync_copy(acc.at[sl], rows0)

        @pl.when(cid == 0)
        def _():
            pltpu.sync_copy(rows0, out0.at[sl])

        @pl.when(cid == 1)
        def _():
            pltpu.sync_copy(rows0, out1.at[sl])


def _sc_scratch():
    t = []
    for _ in range(NSETS):
        t.append(pltpu.VMEM((CHUNK,), jnp.int32))
    for _ in range(NSETS):
        t.append(pltpu.VMEM((CHUNK,), jnp.int32))
    for _ in range(NSETS):
        t.append(pltpu.VMEM((CHUNK, 128), jnp.float32))
    t.append(pltpu.VMEM_SHARED((NPAD, 128), jnp.float32))
    for _ in range(3 * NSETS):
        t.append(pltpu.SemaphoreType.DMA)
    return t


NCH_P = E // (NC * NS * CHUNK)
NCH_C = E // (NS * CHUNK)


@jax.jit
def _sc_pass_partial(table, src, dst):
    @functools.partial(
        pl.kernel, mesh=_vmesh(),
        out_type=[jax.ShapeDtypeStruct((NPAD, 128), jnp.float32),
                  jax.ShapeDtypeStruct((NPAD, 128), jnp.float32)],
        scratch_types=_sc_scratch(),
    )
    def k(table_h, src_h, dst_h, out0_h, out1_h, *scr):
        isrc = scr[0:NSETS]
        idst = scr[NSETS:2 * NSETS]
        rows = scr[2 * NSETS:3 * NSETS]
        acc = scr[3 * NSETS]
        sem_i = scr[3 * NSETS + 1:3 * NSETS + 1 + NSETS]
        sem_g = scr[3 * NSETS + 1 + NSETS:3 * NSETS + 1 + 2 * NSETS]
        sem_s = scr[3 * NSETS + 1 + 2 * NSETS:3 * NSETS + 1 + 3 * NSETS]
        cid = lax.axis_index("c")
        sid = lax.axis_index("s")
        w = cid * NS + sid
        _sc_prologue(acc, rows[0], sid)
        plsc.subcore_barrier()
        _scatter_chunks(table_h, src_h, dst_h, w * (E // (NC * NS)), acc,
                        isrc, idst, rows, sem_i, sem_g, sem_s, NCH_P)
        _sc_epilogue(acc, rows[0], out0_h, out1_h, cid, sid)

    return k(table, src, dst)


@jax.jit
def _sc_pass_colsplit(table_lo, table_hi, src, dst):
    @functools.partial(
        pl.kernel, mesh=_vmesh(),
        out_type=[jax.ShapeDtypeStruct((NPAD, 128), jnp.float32),
                  jax.ShapeDtypeStruct((NPAD, 128), jnp.float32)],
        scratch_types=_sc_scratch(),
    )
    def k(lo_h, hi_h, src_h, dst_h, out0_h, out1_h, *scr):
        isrc = scr[0:NSETS]
        idst = scr[NSETS:2 * NSETS]
        rows = scr[2 * NSETS:3 * NSETS]
        acc = scr[3 * NSETS]
        sem_i = scr[3 * NSETS + 1:3 * NSETS + 1 + NSETS]
        sem_g = scr[3 * NSETS + 1 + NSETS:3 * NSETS + 1 + 2 * NSETS]
        sem_s = scr[3 * NSETS + 1 + 2 * NSETS:3 * NSETS + 1 + 3 * NSETS]
        cid = lax.axis_index("c")
        sid = lax.axis_index("s")
        _sc_prologue(acc, rows[0], sid)
        plsc.subcore_barrier()
        base = sid * (E // NS)

        @pl.when(cid == 0)
        def _():
            _scatter_chunks(lo_h, src_h, dst_h, base, acc, isrc, idst,
                            rows, sem_i, sem_g, sem_s, NCH_C)

        @pl.when(cid == 1)
        def _():
            _scatter_chunks(hi_h, src_h, dst_h, base, acc, isrc, idst,
                            rows, sem_i, sem_g, sem_s, NCH_C)

        _sc_epilogue(acc, rows[0], out0_h, out1_h, cid, sid)

    return k(table_lo, table_hi, src, dst)


RBLK = 2000

def _bdot(a, b):
    return jnp.dot(a.astype(jnp.bfloat16), b.astype(jnp.bfloat16),
                   preferred_element_type=jnp.float32)


def _matvec_body(*refs, nh):
    it = iter(refs)
    h_refs = [next(it) for _ in range(nh)]
    w_refs = [next(it) for _ in range(nh)]
    b_ref = next(it)
    o_ref = next(it)
    acc = b_ref[...].astype(jnp.float32)
    for h_ref, w_ref in zip(h_refs, w_refs):
        acc = acc + _bdot(h_ref[...], w_ref[...])
    o_ref[...] = acc


def _tc_matvec(hs, ws, b):
    nh = len(hs)
    m = ws[0].shape[1]
    in_specs = [pl.BlockSpec((RBLK, h.shape[1]), lambda i: (i, 0))
                for h in hs]
    in_specs += [pl.BlockSpec(w.shape, lambda i: (0, 0)) for w in ws]
    in_specs.append(pl.BlockSpec((1, m), lambda i: (0, 0)))
    return pl.pallas_call(
        functools.partial(_matvec_body, nh=nh),
        grid=(N // RBLK,),
        in_specs=in_specs,
        out_specs=pl.BlockSpec((RBLK, m), lambda i: (i, 0)),
        out_shape=jax.ShapeDtypeStruct((N, m), jnp.float32),
    )(*hs, *ws, b)


def _gru_body(*refs, d, relu, nh, nout, combine, nm):
    it = iter(refs)
    sa_ref, sb_ref, gh_ref = next(it), next(it), next(it)
    h_refs = [next(it) for _ in range(nh)]
    wih_ref = next(it)
    bih_ref = next(it)
    wm_ref = next(it) if nm else None
    o_refs = [next(it) for _ in range(nout + nm)]

    if combine == "add":
        agg = sa_ref[...] + sb_ref[...]
    else:
        agg = jnp.concatenate([sa_ref[...], sb_ref[...]], axis=1)
    gi = _bdot(agg, wih_ref[...]) + bih_ref[...]
    gh = gh_ref[...]
    r = jax.nn.sigmoid(gi[:, :d] + gh[:, :d])
    z = jax.nn.sigmoid(gi[:, d:2 * d] + gh[:, d:2 * d])
    nn = jnp.tanh(gi[:, 2 * d:] + r * gh[:, 2 * d:])
    h = jnp.concatenate([h_ref[...] for h_ref in h_refs], axis=1)
    if h.shape[1] < d:
        h = jnp.concatenate(
            [h, jnp.zeros((h.shape[0], d - h.shape[1]), h.dtype)], axis=1)
    out = (1.0 - z) * nn + z * h
    if relu:
        out = jnp.maximum(out, 0.0)
    if nout == 1:
        o_refs[0][...] = out
    else:
        for i in range(nout):
            o_refs[i][...] = out[:, i * 128:(i + 1) * 128]
    if nm:
        m = _bdot(out, wm_ref[...])
        for i in range(nm):
            o_refs[nout + i][...] = m[:, i * 128:(i + 1) * 128]


def _tc_gru(sa, sb, gh, hs, wih, bih, d, relu, nout, combine, wm=None, nm=0):
    nh = len(hs)
    kdim = 128 if combine == "add" else 256
    in_specs = [
        pl.BlockSpec((RBLK, 128), lambda i: (i, 0)),
        pl.BlockSpec((RBLK, 128), lambda i: (i, 0)),
        pl.BlockSpec((RBLK, 3 * d), lambda i: (i, 0)),
    ]
    for h in hs:
        in_specs.append(pl.BlockSpec((RBLK, h.shape[1]), lambda i: (i, 0)))
    in_specs.append(pl.BlockSpec((kdim, 3 * d), lambda i: (0, 0)))
    in_specs.append(pl.BlockSpec((1, 3 * d), lambda i: (0, 0)))
    if nm:
        in_specs.append(pl.BlockSpec(wm.shape, lambda i: (0, 0)))
    outs = []
    if nout == 1:
        outs.append((pl.BlockSpec((RBLK, d), lambda i: (i, 0)),
                     jax.ShapeDtypeStruct((N, d), jnp.float32)))
    else:
        for _ in range(nout):
            outs.append((pl.BlockSpec((RBLK, 128), lambda i: (i, 0)),
                         jax.ShapeDtypeStruct((N, 128), jnp.float32)))
    for _ in range(nm):
        outs.append((pl.BlockSpec((RBLK, 128), lambda i: (i, 0)),
                     jax.ShapeDtypeStruct((N, 128), jnp.float32)))
    args = [sa, sb, gh, *hs, wih, bih]
    if nm:
        args.append(wm)
    res = pl.pallas_call(
        functools.partial(_gru_body, d=d, relu=relu, nh=nh, nout=nout,
                          combine=combine, nm=nm),
        grid=(N // RBLK,),
        in_specs=in_specs,
        out_specs=[o[0] for o in outs],
        out_shape=[o[1] for o in outs],
    )(*args)
    return res


def _segmax_body(xlo_ref, xhi_ref, b_ref, wf_ref, bf_ref, o_ref):
    b = b_ref[...]
    x = jnp.concatenate([xlo_ref[...], xhi_ref[...]], axis=1)

    def body(g, segs):
        v = jnp.where(b == g, x, -jnp.inf)
        seg = jnp.max(v, axis=0, keepdims=True)
        row_iota = lax.broadcasted_iota(jnp.int32, (G, 1), 0)
        return jnp.where(row_iota == g, seg, segs)

    segs = lax.fori_loop(0, G, body,
                         jnp.full((G, D2), -jnp.inf, jnp.float32))
    o_ref[...] = _bdot(segs, wf_ref[...]) + bf_ref[...]


def _tc_segmax_head(xlo, xhi, batch2d, wf_pad, bf_pad):
    return pl.pallas_call(
        _segmax_body,
        grid=(1,),
        in_specs=[
            pl.BlockSpec((N, 128), lambda g: (0, 0)),
            pl.BlockSpec((N, 128), lambda g: (0, 0)),
            pl.BlockSpec((N, 1), lambda g: (0, 0)),
            pl.BlockSpec((D2, 128), lambda g: (0, 0)),
            pl.BlockSpec((1, 128), lambda g: (0, 0)),
        ],
        out_specs=pl.BlockSpec((G, 128), lambda g: (0, 0)),
        out_shape=jax.ShapeDtypeStruct((G, 128), jnp.float32),
    )(xlo, xhi, batch2d, wf_pad, bf_pad)


def kernel(x, edge_index, batch, weight1, Wih1, Whh1, bih1, bhh1,
           weight2, Wih2, Whh2, bih2, bhh2, Wf, bf):
    src = edge_index[0].astype(jnp.int32)
    dst = edge_index[1].astype(jnp.int32)

    wihT1 = Wih1.T
    wihT2 = Wih2.T
    whhT1 = Whh1.T
    whhT2 = Whh2.T
    bih1r = bih1.reshape(1, -1)
    bhh1r = bhh1.reshape(1, -1)
    bih2r = bih2.reshape(1, -1)
    bhh2r = bhh2.reshape(1, -1)

    m1 = _tc_matvec([x], [weight1[0]], jnp.zeros((1, 128), jnp.float32))
    s1a, s1b = _sc_pass_partial(m1, src, dst)
    gh1 = _tc_matvec([x], [whhT1], bhh1r)
    x1, m2 = _tc_gru(s1a, s1b, gh1, [x], wihT1, bih1r, D1, False, 1,
                     "add", weight1[1], 1)

    s2a, s2b = _sc_pass_partial(m2, src, dst)
    gh2 = _tc_matvec([x1], [whhT1], bhh1r)
    y, m3lo, m3hi = _tc_gru(s2a, s2b, gh2, [x1], wihT1, bih1r, D1, True, 1,
                            "add", weight2[0][:128, :], 2)

    s3a, s3b = _sc_pass_colsplit(m3lo, m3hi, src, dst)
    gh3 = _tc_matvec([y], [whhT2[:128, :]], bhh2r)
    x3lo, x3hi, m4lo, m4hi = _tc_gru(s3a, s3b, gh3, [y], wihT2, bih2r,
                                     D2, False, 2, "concat",
                                     weight2[1], 2)

    s4a, s4b = _sc_pass_colsplit(m4lo, m4hi, src, dst)
    gh4 = _tc_matvec([x3lo, x3hi], [whhT2[:128, :], whhT2[128:, :]], bhh2r)
    x4lo, x4hi = _tc_gru(s4a, s4b, gh4, [x3lo, x3hi], wihT2, bih2r,
                         D2, False, 2, "concat")

    batch2d = batch.astype(jnp.int32).reshape(N, 1)
    wf_pad = jnp.zeros((D2, 128), jnp.float32).at[:, :6].set(Wf.T)
    bf_pad = jnp.zeros((1, 128), jnp.float32).at[0, :6].set(bf)
    out = _tc_segmax_head(x4lo, x4hi, batch2d, wf_pad, bf_pad)
    return out[:, :6]
